# Initial kernel scaffold; baseline (speedup 1.0000x reference)
import jax
import jax.numpy as jnp
from jax.experimental import pallas as pl


def kernel(input_ids, word_table, pos_table, gamma, beta):
    raise NotImplementedError("write your pallas kernel here")


# speedup vs baseline: 2.3230x; 2.3230x over previous
import functools

import jax
import jax.numpy as jnp
from jax import lax
from jax.experimental import pallas as pl
from jax.experimental.pallas import tpu as pltpu
from jax.experimental.pallas import tpu_sc as plsc

B, L = 1024, 200
H = 128
N = B * L
NC, NS = 2, 16
NW = NC * NS
TOK_PER_W = N // NW
CHUNK = 128
NCHUNK = TOK_PER_W // CHUNK
EPS = 1e-12
NVEC = H // 16


def _rsqrt(x):
    i = lax.bitcast_convert_type(x, jnp.int32)
    i = jnp.int32(0x5F3759DF) - lax.shift_right_arithmetic(i, 1)
    y = lax.bitcast_convert_type(i, jnp.float32)
    half_x = 0.5 * x
    for _ in range(3):
        y = y * (1.5 - half_x * y * y)
    return y


def _hsum(x):
    iota = lax.iota(jnp.int32, 16)
    for k in (8, 4, 2, 1):
        x = x + x.at[lax.bitwise_xor(iota, k)].get(mode="promise_in_bounds")
    return x


def _token_layernorm(j, in_ref, out_ref, pos_ref, gamma_ref, beta_ref, cbase):
    p = lax.rem(cbase + j, L)
    xs = []
    for h in range(NVEC):
        w = in_ref[j, pl.ds(h * 16, 16)]
        pv = pos_ref[p, pl.ds(h * 16, 16)]
        xs.append(w + pv)
    t0 = (xs[0] + xs[1]) + (xs[2] + xs[3])
    t1 = (xs[4] + xs[5]) + (xs[6] + xs[7])
    mean = _hsum(t0 + t1) * (1.0 / H)
    ds_ = [x - mean for x in xs]
    q = [d * d for d in ds_]
    v0 = (q[0] + q[1]) + (q[2] + q[3])
    v1 = (q[4] + q[5]) + (q[6] + q[7])
    var = _hsum(v0 + v1) * (1.0 / H)
    rstd = _rsqrt(var + EPS)
    for h in range(NVEC):
        g = gamma_ref[pl.ds(h * 16, 16)]
        bb = beta_ref[pl.ds(h * 16, 16)]
        out_ref[j, pl.ds(h * 16, 16)] = ds_[h] * rstd * g + bb


def _sc_body(ids_hbm, wtab_hbm, ptab_hbm, gamma_hbm, beta_hbm, out_hbm,
             pos_v, gamma_v, beta_v,
             idx0, idx1, in0, in1, out0, out1,
             g0, g1, s0, s1):
    wid = lax.axis_index("s") * NC + lax.axis_index("c")
    base = wid * TOK_PER_W

    pltpu.sync_copy(ptab_hbm.at[pl.ds(0, L)], pos_v)
    pltpu.sync_copy(gamma_hbm, gamma_v)
    pltpu.sync_copy(beta_hbm, beta_v)

    idxs = (idx0, idx1)
    ins = (in0, in1)
    outs = (out0, out1)
    gsems = (g0, g1)
    ssems = (s0, s1)

    for b in range(2):
        pltpu.sync_copy(ids_hbm.at[pl.ds(base + b * CHUNK, CHUNK)], idxs[b])
        pltpu.async_copy(wtab_hbm.at[idxs[b]], ins[b], gsems[b])

    @pl.loop(0, NCHUNK // 2)
    def _superiter(gi):
        for b in range(2):
            c = gi * 2 + b
            tok = base + c * CHUNK
            cbase = lax.rem(c * CHUNK, L)
            pltpu.make_async_copy(
                wtab_hbm.at[pl.ds(0, CHUNK)], ins[b], gsems[b]).wait()

            @pl.when(gi > 0)
            def _():
                pltpu.make_async_copy(
                    outs[b], out_hbm.at[pl.ds(tok - 2 * CHUNK, CHUNK)],
                    ssems[b]).wait()

            @pl.loop(0, CHUNK)
            def _tok(j):
                _token_layernorm(j, ins[b], outs[b], pos_v, gamma_v, beta_v,
                                 cbase)

            @pl.when(c + 2 < NCHUNK)
            def _():
                pltpu.sync_copy(
                    ids_hbm.at[pl.ds(tok + 2 * CHUNK, CHUNK)], idxs[b])
                pltpu.async_copy(wtab_hbm.at[idxs[b]], ins[b], gsems[b])

            pltpu.async_copy(outs[b], out_hbm.at[pl.ds(tok, CHUNK)], ssems[b])

    for b in range(2):
        tokl = base + (NCHUNK - 2 + b) * CHUNK
        pltpu.make_async_copy(
            outs[b], out_hbm.at[pl.ds(tokl, CHUNK)], ssems[b]).wait()


_sc_kernel = pl.kernel(
    _sc_body,
    out_type=jax.ShapeDtypeStruct((N, H), jnp.float32),
    mesh=plsc.VectorSubcoreMesh(
        core_axis_name="c", subcore_axis_name="s",
        num_cores=NC, num_subcores=NS),
    scratch_types=[
        pltpu.VMEM((L, H), jnp.float32),
        pltpu.VMEM((H,), jnp.float32),
        pltpu.VMEM((H,), jnp.float32),
        pltpu.VMEM((CHUNK,), jnp.int32),
        pltpu.VMEM((CHUNK,), jnp.int32),
        pltpu.VMEM((CHUNK, H), jnp.float32),
        pltpu.VMEM((CHUNK, H), jnp.float32),
        pltpu.VMEM((CHUNK, H), jnp.float32),
        pltpu.VMEM((CHUNK, H), jnp.float32),
        pltpu.SemaphoreType.DMA,
        pltpu.SemaphoreType.DMA,
        pltpu.SemaphoreType.DMA,
        pltpu.SemaphoreType.DMA,
    ],
)


@jax.jit
def kernel(input_ids, word_table, pos_table, gamma, beta):
    ids_flat = input_ids.reshape(N).astype(jnp.int32)
    out = _sc_kernel(ids_flat, word_table, pos_table, gamma, beta)
    return out.reshape(B, L, H)


# speedup vs baseline: 2.4805x; 1.0678x over previous
import functools

import jax
import jax.numpy as jnp
from jax import lax
from jax.experimental import pallas as pl
from jax.experimental.pallas import tpu as pltpu
from jax.experimental.pallas import tpu_sc as plsc

B, L = 1024, 200
H = 128
N = B * L
NC, NS = 2, 16
NW = NC * NS
TOK_PER_W = N // NW
CHUNK = 128
NCHUNK = TOK_PER_W // CHUNK
EPS = 1e-12
NVEC = H // 16


def _rsqrt(x):
    i = lax.bitcast_convert_type(x, jnp.int32)
    i = jnp.int32(0x5F3759DF) - lax.shift_right_arithmetic(i, 1)
    y = lax.bitcast_convert_type(i, jnp.float32)
    half_x = 0.5 * x
    for _ in range(3):
        y = y * (1.5 - half_x * y * y)
    return y


def _hsum(x):
    iota = lax.iota(jnp.int32, 16)
    for k in (8, 4, 2, 1):
        x = x + x.at[lax.bitwise_xor(iota, k)].get(mode="promise_in_bounds")
    return x


def _token_layernorm(j, in_ref, out_ref, pos_ref, gamma_ref, beta_ref, cbase):
    p = lax.rem(cbase + j, L)
    xs = []
    for h in range(NVEC):
        w = in_ref[j, pl.ds(h * 16, 16)]
        pv = pos_ref[p, pl.ds(h * 16, 16)]
        xs.append(w + pv)
    t0 = (xs[0] + xs[1]) + (xs[2] + xs[3])
    t1 = (xs[4] + xs[5]) + (xs[6] + xs[7])
    q = [x * x for x in xs]
    q0 = (q[0] + q[1]) + (q[2] + q[3])
    q1 = (q[4] + q[5]) + (q[6] + q[7])
    mean = _hsum(t0 + t1) * (1.0 / H)
    ex2 = _hsum(q0 + q1) * (1.0 / H)
    var = ex2 - mean * mean
    rstd = _rsqrt(var + EPS)
    for h in range(NVEC):
        s2 = rstd * gamma_ref[pl.ds(h * 16, 16)]
        bb = beta_ref[pl.ds(h * 16, 16)]
        out_ref[j, pl.ds(h * 16, 16)] = (xs[h] - mean) * s2 + bb


def _sc_body(ids_hbm, wtab_hbm, ptab_hbm, gamma_hbm, beta_hbm, out_hbm,
             pos_v, gamma_v, beta_v,
             idx0, idx1, in0, in1, out0, out1,
             g0, g1, s0, s1):
    wid = lax.axis_index("s") * NC + lax.axis_index("c")
    base = wid * TOK_PER_W

    pltpu.sync_copy(ptab_hbm.at[pl.ds(0, L)], pos_v)
    pltpu.sync_copy(gamma_hbm, gamma_v)
    pltpu.sync_copy(beta_hbm, beta_v)

    idxs = (idx0, idx1)
    ins = (in0, in1)
    outs = (out0, out1)
    gsems = (g0, g1)
    ssems = (s0, s1)

    for b in range(2):
        pltpu.sync_copy(ids_hbm.at[pl.ds(base + b * CHUNK, CHUNK)], idxs[b])
        pltpu.async_copy(wtab_hbm.at[idxs[b]], ins[b], gsems[b])

    @pl.loop(0, NCHUNK // 2)
    def _superiter(gi):
        for b in range(2):
            c = gi * 2 + b
            tok = base + c * CHUNK
            cbase = lax.rem(c * CHUNK, L)
            pltpu.make_async_copy(
                wtab_hbm.at[pl.ds(0, CHUNK)], ins[b], gsems[b]).wait()

            @pl.when(gi > 0)
            def _():
                pltpu.make_async_copy(
                    outs[b], out_hbm.at[pl.ds(tok - 2 * CHUNK, CHUNK)],
                    ssems[b]).wait()

            @plsc.parallel_loop(0, CHUNK, unroll=4)
            def _tok(j):
                _token_layernorm(j, ins[b], outs[b], pos_v, gamma_v, beta_v,
                                 cbase)

            @pl.when(c + 2 < NCHUNK)
            def _():
                pltpu.sync_copy(
                    ids_hbm.at[pl.ds(tok + 2 * CHUNK, CHUNK)], idxs[b])
                pltpu.async_copy(wtab_hbm.at[idxs[b]], ins[b], gsems[b])

            pltpu.async_copy(outs[b], out_hbm.at[pl.ds(tok, CHUNK)], ssems[b])

    for b in range(2):
        tokl = base + (NCHUNK - 2 + b) * CHUNK
        pltpu.make_async_copy(
            outs[b], out_hbm.at[pl.ds(tokl, CHUNK)], ssems[b]).wait()


_sc_kernel = pl.kernel(
    _sc_body,
    out_type=jax.ShapeDtypeStruct((N, H), jnp.float32),
    mesh=plsc.VectorSubcoreMesh(
        core_axis_name="c", subcore_axis_name="s",
        num_cores=NC, num_subcores=NS),
    scratch_types=[
        pltpu.VMEM((L, H), jnp.float32),
        pltpu.VMEM((H,), jnp.float32),
        pltpu.VMEM((H,), jnp.float32),
        pltpu.VMEM((CHUNK,), jnp.int32),
        pltpu.VMEM((CHUNK,), jnp.int32),
        pltpu.VMEM((CHUNK, H), jnp.float32),
        pltpu.VMEM((CHUNK, H), jnp.float32),
        pltpu.VMEM((CHUNK, H), jnp.float32),
        pltpu.VMEM((CHUNK, H), jnp.float32),
        pltpu.SemaphoreType.DMA,
        pltpu.SemaphoreType.DMA,
        pltpu.SemaphoreType.DMA,
        pltpu.SemaphoreType.DMA,
    ],
)


@jax.jit
def kernel(input_ids, word_table, pos_table, gamma, beta):
    ids_flat = input_ids.reshape(N).astype(jnp.int32)
    out = _sc_kernel(ids_flat, word_table, pos_table, gamma, beta)
    return out.reshape(B, L, H)


# speedup vs baseline: 7.7704x; 3.1326x over previous
import functools

import jax
import jax.numpy as jnp
from jax import lax
from jax.experimental import pallas as pl
from jax.experimental.pallas import tpu as pltpu
from jax.experimental.pallas import tpu_sc as plsc

B, L = 1024, 200
H = 128
N = B * L
NC, NS = 2, 16
NW = NC * NS
TOK_PER_W = N // NW
CHUNK = 128
NCHUNK = TOK_PER_W // CHUNK
EPS = 1e-12
NVEC = H // 16


def _rsqrt(x):
    i = lax.bitcast_convert_type(x, jnp.int32)
    i = jnp.int32(0x5F3759DF) - lax.shift_right_arithmetic(i, 1)
    y = lax.bitcast_convert_type(i, jnp.float32)
    half_x = 0.5 * x
    for _ in range(3):
        y = y * (1.5 - half_x * y * y)
    return y


def _hsum(x):
    iota = lax.iota(jnp.int32, 16)
    for k in (8, 4, 2, 1):
        x = x + x.at[lax.bitwise_xor(iota, k)].get(mode="promise_in_bounds")
    return x


def _token_layernorm(j, in_ref, out_ref, pos_ref, gs, bs, cbase):
    p = lax.rem(cbase + j, L)
    x = in_ref[j, pl.ds(0, 16)] + pos_ref[p, pl.ds(0, 16)]
    out_ref[j, pl.ds(0, 16)] = x
    s = x
    q = x * x
    for h in range(1, NVEC):
        x = in_ref[j, pl.ds(h * 16, 16)] + pos_ref[p, pl.ds(h * 16, 16)]
        out_ref[j, pl.ds(h * 16, 16)] = x
        s = s + x
        q = q + x * x
    mean = _hsum(s) * (1.0 / H)
    ex2 = _hsum(q) * (1.0 / H)
    var = ex2 - mean * mean
    rstd = _rsqrt(var + EPS)
    for h in range(NVEC):
        n = (out_ref[j, pl.ds(h * 16, 16)] - mean) * rstd
        out_ref[j, pl.ds(h * 16, 16)] = n * gs[h] + bs[h]


def _sc_body(ids_hbm, wtab_hbm, ptab_hbm, gamma_hbm, beta_hbm, out_hbm,
             pos_v, gamma_v, beta_v,
             idx0, idx1, in0, in1, out0, out1,
             g0, g1, s0, s1):
    wid = lax.axis_index("s") * NC + lax.axis_index("c")
    base = wid * TOK_PER_W

    pltpu.sync_copy(ptab_hbm.at[pl.ds(0, L)], pos_v)
    pltpu.sync_copy(gamma_hbm, gamma_v)
    pltpu.sync_copy(beta_hbm, beta_v)

    idxs = (idx0, idx1)
    ins = (in0, in1)
    outs = (out0, out1)
    gsems = (g0, g1)
    ssems = (s0, s1)

    for b in range(2):
        pltpu.sync_copy(ids_hbm.at[pl.ds(base + b * CHUNK, CHUNK)], idxs[b])
        pltpu.async_copy(wtab_hbm.at[idxs[b]], ins[b], gsems[b])

    gs = [gamma_v[pl.ds(h * 16, 16)] for h in range(NVEC)]
    bs = [beta_v[pl.ds(h * 16, 16)] for h in range(NVEC)]

    @pl.loop(0, NCHUNK // 2)
    def _superiter(gi):
        for b in range(2):
            c = gi * 2 + b
            tok = base + c * CHUNK
            cbase = lax.rem(c * CHUNK, L)
            pltpu.make_async_copy(
                wtab_hbm.at[pl.ds(0, CHUNK)], ins[b], gsems[b]).wait()

            @pl.when(gi > 0)
            def _():
                pltpu.make_async_copy(
                    outs[b], out_hbm.at[pl.ds(tok - 2 * CHUNK, CHUNK)],
                    ssems[b]).wait()

            @plsc.parallel_loop(0, CHUNK, unroll=4)
            def _tok(j):
                _token_layernorm(j, ins[b], outs[b], pos_v, gs, bs, cbase)

            @pl.when(c + 2 < NCHUNK)
            def _():
                pltpu.sync_copy(
                    ids_hbm.at[pl.ds(tok + 2 * CHUNK, CHUNK)], idxs[b])
                pltpu.async_copy(wtab_hbm.at[idxs[b]], ins[b], gsems[b])

            pltpu.async_copy(outs[b], out_hbm.at[pl.ds(tok, CHUNK)], ssems[b])

    for b in range(2):
        tokl = base + (NCHUNK - 2 + b) * CHUNK
        pltpu.make_async_copy(
            outs[b], out_hbm.at[pl.ds(tokl, CHUNK)], ssems[b]).wait()


_sc_kernel = pl.kernel(
    _sc_body,
    out_type=jax.ShapeDtypeStruct((N, H), jnp.float32),
    mesh=plsc.VectorSubcoreMesh(
        core_axis_name="c", subcore_axis_name="s",
        num_cores=NC, num_subcores=NS),
    scratch_types=[
        pltpu.VMEM((L, H), jnp.float32),
        pltpu.VMEM((H,), jnp.float32),
        pltpu.VMEM((H,), jnp.float32),
        pltpu.VMEM((CHUNK,), jnp.int32),
        pltpu.VMEM((CHUNK,), jnp.int32),
        pltpu.VMEM((CHUNK, H), jnp.float32),
        pltpu.VMEM((CHUNK, H), jnp.float32),
        pltpu.VMEM((CHUNK, H), jnp.float32),
        pltpu.VMEM((CHUNK, H), jnp.float32),
        pltpu.SemaphoreType.DMA,
        pltpu.SemaphoreType.DMA,
        pltpu.SemaphoreType.DMA,
        pltpu.SemaphoreType.DMA,
    ],
)


@jax.jit
def kernel(input_ids, word_table, pos_table, gamma, beta):
    ids_flat = input_ids.reshape(N).astype(jnp.int32)
    out = _sc_kernel(ids_flat, word_table, pos_table, gamma, beta)
    return out.reshape(B, L, H)


# speedup vs baseline: 8.1618x; 1.0504x over previous
import functools

import jax
import jax.numpy as jnp
from jax import lax
from jax.experimental import pallas as pl
from jax.experimental.pallas import tpu as pltpu
from jax.experimental.pallas import tpu_sc as plsc

B, L = 1024, 200
H = 128
N = B * L
NC, NS = 2, 16
NW = NC * NS
TOK_PER_W = N // NW
CHUNK = 128
NCHUNK = TOK_PER_W // CHUNK
NBUF = 5
AHEAD = 2
EPS = 1e-12
BB = 16


def _sc_gather_body(ids_hbm, wtab_hbm, out_hbm, idxall, *bufs_and_sems):
    bufs = bufs_and_sems[:NBUF]
    gsems = bufs_and_sems[NBUF:2 * NBUF]
    ssems = bufs_and_sems[2 * NBUF:]
    wid = lax.axis_index("s") * NC + lax.axis_index("c")
    base = wid * TOK_PER_W

    pltpu.sync_copy(ids_hbm.at[wid], idxall)

    for c in range(AHEAD):
        pltpu.async_copy(wtab_hbm.at[idxall.at[c]], bufs[c], gsems[c])

    @pl.loop(0, NCHUNK // NBUF)
    def _superiter(gi):
        for b in range(NBUF):
            c = gi * NBUF + b
            pltpu.make_async_copy(
                wtab_hbm.at[pl.ds(0, CHUNK)], bufs[b], gsems[b]).wait()

            b2 = (b + AHEAD) % NBUF

            @pl.when(c + AHEAD < NCHUNK)
            def _():
                @pl.when(c + AHEAD >= NBUF)
                def _():
                    pltpu.make_async_copy(
                        bufs[b2], out_hbm.at[pl.ds(base, CHUNK)],
                        ssems[b2]).wait()
                pltpu.async_copy(
                    wtab_hbm.at[idxall.at[c + AHEAD]], bufs[b2], gsems[b2])

            pltpu.async_copy(
                bufs[b], out_hbm.at[pl.ds(base + c * CHUNK, CHUNK)],
                ssems[b])

    for c in range(NCHUNK - NBUF, NCHUNK):
        b = c % NBUF
        pltpu.make_async_copy(
            bufs[b], out_hbm.at[pl.ds(base + c * CHUNK, CHUNK)],
            ssems[b]).wait()


_sc_gather = pl.kernel(
    _sc_gather_body,
    out_type=jax.ShapeDtypeStruct((N, H), jnp.float32),
    mesh=plsc.VectorSubcoreMesh(
        core_axis_name="c", subcore_axis_name="s",
        num_cores=NC, num_subcores=NS),
    scratch_types=(
        [pltpu.VMEM((NCHUNK, CHUNK), jnp.int32)]
        + [pltpu.VMEM((CHUNK, H), jnp.float32) for _ in range(NBUF)]
        + [pltpu.SemaphoreType.DMA for _ in range(2 * NBUF)]
    ),
)


def _tc_ln_body(x_ref, pos_ref, g_ref, b_ref, o_ref):
    x = x_ref[...] + pos_ref[...][None, :, :]
    mean = jnp.mean(x, axis=-1, keepdims=True)
    xc = x - mean
    var = jnp.mean(xc * xc, axis=-1, keepdims=True)
    normed = xc * lax.rsqrt(var + EPS)
    o_ref[...] = normed * g_ref[0][None, None, :] + b_ref[0][None, None, :]


_tc_ln = pl.pallas_call(
    _tc_ln_body,
    grid=(B // BB,),
    in_specs=[
        pl.BlockSpec((BB, L, H), lambda i: (i, 0, 0)),
        pl.BlockSpec((L, H), lambda i: (0, 0)),
        pl.BlockSpec((1, H), lambda i: (0, 0)),
        pl.BlockSpec((1, H), lambda i: (0, 0)),
    ],
    out_specs=pl.BlockSpec((BB, L, H), lambda i: (i, 0, 0)),
    out_shape=jax.ShapeDtypeStruct((B, L, H), jnp.float32),
)


@jax.jit
def kernel(input_ids, word_table, pos_table, gamma, beta):
    ids_flat = input_ids.reshape(NW, NCHUNK, CHUNK).astype(jnp.int32)
    gathered = _sc_gather(ids_flat, word_table)
    return _tc_ln(
        gathered.reshape(B, L, H),
        pos_table[:L],
        gamma.reshape(1, H),
        beta.reshape(1, H),
    )


# speedup vs baseline: 8.9853x; 1.1009x over previous
import functools

import jax
import jax.numpy as jnp
from jax import lax
from jax.experimental import pallas as pl
from jax.experimental.pallas import tpu as pltpu
from jax.experimental.pallas import tpu_sc as plsc

B, L = 1024, 200
H = 128
N = B * L
NC, NS = 2, 16
NW = NC * NS
TOK_PER_W = N // NW
CHUNK = 128
NCHUNK = TOK_PER_W // CHUNK
NBUF = 5
AHEAD = 2
EPS = 1e-12
BB = 32


def _sc_gather_body(ids_hbm, wtab_hbm, out_hbm, idxall, *bufs_and_sems):
    bufs = bufs_and_sems[:NBUF]
    gsems = bufs_and_sems[NBUF:2 * NBUF]
    ssems = bufs_and_sems[2 * NBUF:]
    wid = lax.axis_index("s") * NC + lax.axis_index("c")
    base = wid * TOK_PER_W

    pltpu.sync_copy(ids_hbm.at[wid], idxall)

    for c in range(AHEAD):
        pltpu.async_copy(wtab_hbm.at[idxall.at[c]], bufs[c], gsems[c])

    @pl.loop(0, NCHUNK // NBUF)
    def _superiter(gi):
        for b in range(NBUF):
            c = gi * NBUF + b
            pltpu.make_async_copy(
                wtab_hbm.at[pl.ds(0, CHUNK)], bufs[b], gsems[b]).wait()

            b2 = (b + AHEAD) % NBUF

            @pl.when(c + AHEAD < NCHUNK)
            def _():
                @pl.when(c + AHEAD >= NBUF)
                def _():
                    pltpu.make_async_copy(
                        bufs[b2], out_hbm.at[pl.ds(base, CHUNK)],
                        ssems[b2]).wait()
                pltpu.async_copy(
                    wtab_hbm.at[idxall.at[c + AHEAD]], bufs[b2], gsems[b2])

            pltpu.async_copy(
                bufs[b], out_hbm.at[pl.ds(base + c * CHUNK, CHUNK)],
                ssems[b])

    for c in range(NCHUNK - NBUF, NCHUNK):
        b = c % NBUF
        pltpu.make_async_copy(
            bufs[b], out_hbm.at[pl.ds(base + c * CHUNK, CHUNK)],
            ssems[b]).wait()


_sc_gather = pl.kernel(
    _sc_gather_body,
    out_type=jax.ShapeDtypeStruct((N, H), jnp.float32),
    mesh=plsc.VectorSubcoreMesh(
        core_axis_name="c", subcore_axis_name="s",
        num_cores=NC, num_subcores=NS),
    scratch_types=(
        [pltpu.VMEM((NCHUNK, CHUNK), jnp.int32)]
        + [pltpu.VMEM((CHUNK, H), jnp.float32) for _ in range(NBUF)]
        + [pltpu.SemaphoreType.DMA for _ in range(2 * NBUF)]
    ),
)


def _tc_ln_body(x_ref, pos_ref, g_ref, b_ref, o_ref):
    x = x_ref[...] + pos_ref[...][None, :, :]
    mean = jnp.mean(x, axis=-1, keepdims=True)
    xc = x - mean
    var = jnp.mean(xc * xc, axis=-1, keepdims=True)
    normed = xc * lax.rsqrt(var + EPS)
    o_ref[...] = normed * g_ref[0][None, None, :] + b_ref[0][None, None, :]


_tc_ln = pl.pallas_call(
    _tc_ln_body,
    grid=(B // BB,),
    in_specs=[
        pl.BlockSpec((BB, L, H), lambda i: (i, 0, 0)),
        pl.BlockSpec((L, H), lambda i: (0, 0)),
        pl.BlockSpec((1, H), lambda i: (0, 0)),
        pl.BlockSpec((1, H), lambda i: (0, 0)),
    ],
    out_specs=pl.BlockSpec((BB, L, H), lambda i: (i, 0, 0)),
    out_shape=jax.ShapeDtypeStruct((B, L, H), jnp.float32),
)


@jax.jit
def kernel(input_ids, word_table, pos_table, gamma, beta):
    ids_flat = input_ids.reshape(NW, NCHUNK, CHUNK).astype(jnp.int32)
    gathered = _sc_gather(ids_flat, word_table)
    return _tc_ln(
        gathered.reshape(B, L, H),
        pos_table[:L],
        gamma.reshape(1, H),
        beta.reshape(1, H),
    )


# speedup vs baseline: 9.4722x; 1.0542x over previous
import functools

import jax
import jax.numpy as jnp
from jax import lax
from jax.experimental import pallas as pl
from jax.experimental.pallas import tpu as pltpu
from jax.experimental.pallas import tpu_sc as plsc

B, L = 1024, 200
H = 128
N = B * L
NC, NS = 2, 16
NW = NC * NS
TOK_PER_W = N // NW
CHUNK = 128
NCHUNK = TOK_PER_W // CHUNK
NBUF = 5
AHEAD = 2
EPS = 1e-12
BB = 64


def _sc_gather_body(ids_hbm, wtab_hbm, out_hbm, idxall, *bufs_and_sems):
    bufs = bufs_and_sems[:NBUF]
    gsems = bufs_and_sems[NBUF:2 * NBUF]
    ssems = bufs_and_sems[2 * NBUF:]
    wid = lax.axis_index("s") * NC + lax.axis_index("c")
    base = wid * TOK_PER_W

    pltpu.sync_copy(ids_hbm.at[wid], idxall)

    for c in range(AHEAD):
        pltpu.async_copy(wtab_hbm.at[idxall.at[c]], bufs[c], gsems[c])

    @pl.loop(0, NCHUNK // NBUF)
    def _superiter(gi):
        for b in range(NBUF):
            c = gi * NBUF + b
            pltpu.make_async_copy(
                wtab_hbm.at[pl.ds(0, CHUNK)], bufs[b], gsems[b]).wait()

            b2 = (b + AHEAD) % NBUF

            @pl.when(c + AHEAD < NCHUNK)
            def _():
                @pl.when(c + AHEAD >= NBUF)
                def _():
                    pltpu.make_async_copy(
                        bufs[b2], out_hbm.at[pl.ds(base, CHUNK)],
                        ssems[b2]).wait()
                pltpu.async_copy(
                    wtab_hbm.at[idxall.at[c + AHEAD]], bufs[b2], gsems[b2])

            pltpu.async_copy(
                bufs[b], out_hbm.at[pl.ds(base + c * CHUNK, CHUNK)],
                ssems[b])

    for c in range(NCHUNK - NBUF, NCHUNK):
        b = c % NBUF
        pltpu.make_async_copy(
            bufs[b], out_hbm.at[pl.ds(base + c * CHUNK, CHUNK)],
            ssems[b]).wait()


_sc_gather = pl.kernel(
    _sc_gather_body,
    out_type=jax.ShapeDtypeStruct((N, H), jnp.float32),
    mesh=plsc.VectorSubcoreMesh(
        core_axis_name="c", subcore_axis_name="s",
        num_cores=NC, num_subcores=NS),
    scratch_types=(
        [pltpu.VMEM((NCHUNK, CHUNK), jnp.int32)]
        + [pltpu.VMEM((CHUNK, H), jnp.float32) for _ in range(NBUF)]
        + [pltpu.SemaphoreType.DMA for _ in range(2 * NBUF)]
    ),
)


def _tc_ln_body(x_ref, pos_ref, g_ref, b_ref, o_ref):
    x = x_ref[...] + pos_ref[...][None, :, :]
    mean = jnp.mean(x, axis=-1, keepdims=True)
    xc = x - mean
    var = jnp.mean(xc * xc, axis=-1, keepdims=True)
    normed = xc * lax.rsqrt(var + EPS)
    o_ref[...] = normed * g_ref[0][None, None, :] + b_ref[0][None, None, :]


_tc_ln = pl.pallas_call(
    _tc_ln_body,
    grid=(B // BB,),
    in_specs=[
        pl.BlockSpec((BB, L, H), lambda i: (i, 0, 0)),
        pl.BlockSpec((L, H), lambda i: (0, 0)),
        pl.BlockSpec((1, H), lambda i: (0, 0)),
        pl.BlockSpec((1, H), lambda i: (0, 0)),
    ],
    out_specs=pl.BlockSpec((BB, L, H), lambda i: (i, 0, 0)),
    out_shape=jax.ShapeDtypeStruct((B, L, H), jnp.float32),
)


@jax.jit
def kernel(input_ids, word_table, pos_table, gamma, beta):
    ids_flat = input_ids.reshape(NW, NCHUNK, CHUNK).astype(jnp.int32)
    gathered = _sc_gather(ids_flat, word_table)
    return _tc_ln(
        gathered.reshape(B, L, H),
        pos_table[:L],
        gamma.reshape(1, H),
        beta.reshape(1, H),
    )
